# baseline (speedup 1.0000x reference)
import functools

import jax
import jax.numpy as jnp
from jax import lax
from jax.experimental import pallas as pl
from jax.experimental.pallas import tpu as pltpu
from jax.experimental.pallas import tpu_sc as plsc

N_NODES = 10000
N_EDGES = 320000
D = 128
N_LAYERS = 3

NC = 2
NS = 16
DH = D // NC
CHUNK = 128
E_PER_T_PAD = 20480
NCHUNK = E_PER_T_PAD // CHUNK
NBUF = 4
ROWS_PER_TILE = N_NODES // NS
ZROWS = 125

_BCAST_DNUMS = lax.GatherDimensionNumbers(
    offset_dims=(), collapsed_slice_dims=(0,), start_index_map=(0,))


def _bcast(v16, i, zero):
  idx = jnp.full((16, 1), zero + i, jnp.int32)
  return lax.gather(v16, idx, _BCAST_DNUMS, (1,),
                    mode=lax.GatherScatterMode.PROMISE_IN_BOUNDS)


def _scale_chunk(rows_v, w2_v, p, b):
  def group(g, carry):
    gbase = pl.multiple_of(g * 16, 16)
    w16 = w2_v[p, b, pl.ds(gbase, 16)]
    zero = g * 0
    for i in range(16):
      wb = _bcast(w16, i, zero)
      e = gbase + i
      for col in range(DH // 16):
        x = rows_v[e, pl.ds(col * 16, 16)]
        rows_v[e, pl.ds(col * 16, 16)] = x * wb
    return carry

  lax.fori_loop(0, CHUNK // 16, group, 0)


NOUTER = NCHUNK // NBUF


def _sc_layers_body(table_hbm, idx_hbm, w_hbm, out_hbm,
                    sidx_v, w2_v, bufs, gsems, esems, fsems, zbuf_v, acc_sh):
  c = lax.axis_index("c")
  s = lax.axis_index("s")
  zeros16 = jnp.zeros((16,), jnp.float32)

  def zero_row(r, carry):
    for col in range(DH // 16):
      zbuf_v[r, pl.ds(col * 16, 16)] = zeros16
    return carry

  lax.fori_loop(0, ZROWS, zero_row, 0)

  def edata_copy(i, p):
    a = pltpu.async_copy(idx_hbm.at[s, pl.ds(i * NBUF, NBUF)],
                         sidx_v.at[p], esems[p])
    bb = pltpu.async_copy(w_hbm.at[s, pl.ds(i * NBUF, NBUF)],
                          w2_v.at[p], fsems[p])
    return a, bb

  def edata_wait(i, p):
    a, bb = pltpu.make_async_copy(idx_hbm.at[s, pl.ds(i * NBUF, NBUF)],
                                  sidx_v.at[p], esems[p]), \
            pltpu.make_async_copy(w_hbm.at[s, pl.ds(i * NBUF, NBUF)],
                                  w2_v.at[p], fsems[p])
    a.wait()
    bb.wait()

  def gather_issue(table, p, b):
    pltpu.async_copy(table.at[c].at[sidx_v.at[p, b, 0]], bufs[b], gsems[b])

  def gather_wait(table, p, b):
    pltpu.make_async_copy(table.at[c].at[sidx_v.at[p, b, 0]],
                          bufs[b], gsems[b]).wait()

  def one_layer(table, lyr):
    for t in range(ROWS_PER_TILE // ZROWS):
      pltpu.sync_copy(zbuf_v, acc_sh.at[pl.ds(s * ROWS_PER_TILE + t * ZROWS,
                                              ZROWS)])
    plsc.subcore_barrier()

    a, bcp = edata_copy(0, 0)
    a.wait()
    bcp.wait()
    for b in range(NBUF):
      gather_issue(table, 0, b)
    edata_copy(1, 1)

    def sub_iter(i, p):
      p1 = 1 - p

      @pl.when(i + 1 < NOUTER)
      def _():
        edata_wait(i + 1, p1)

      for b in range(NBUF):
        gather_wait(table, p, b)
        _scale_chunk(bufs[b], w2_v, p, b)
        pltpu.sync_copy(bufs[b], acc_sh.at[sidx_v.at[p, b, 1]], add=True)

        @pl.when(i + 1 < NOUTER)
        def _():
          gather_issue(table, p1, b)

      @pl.when(i + 2 < NOUTER)
      def _():
        edata_copy(i + 2, p)

    def outer(io, carry):
      sub_iter(io * 2, 0)
      sub_iter(io * 2 + 1, 1)
      return carry

    lax.fori_loop(0, NOUTER // 2, outer, 0)
    plsc.subcore_barrier()

    for t in range(ROWS_PER_TILE // ZROWS):
      r0 = s * ROWS_PER_TILE + t * ZROWS
      pltpu.sync_copy(acc_sh.at[pl.ds(r0, ZROWS)],
                      out_hbm.at[c, lyr, pl.ds(r0, ZROWS)])
    plsc.subcore_barrier()

  one_layer(table_hbm, 0)
  one_layer(out_hbm.at[:, 0], 1)
  one_layer(out_hbm.at[:, 1], 2)


@functools.cache
def _get_sc_layers():
  return pl.kernel(
      _sc_layers_body,
      out_type=jax.ShapeDtypeStruct((NC, N_LAYERS, N_NODES, DH), jnp.float32),
      mesh=plsc.VectorSubcoreMesh(core_axis_name="c", subcore_axis_name="s",
                                  num_cores=NC, num_subcores=NS),
      scratch_types=[
          pltpu.VMEM((2, NBUF, 2, CHUNK), jnp.int32),
          pltpu.VMEM((2, NBUF, CHUNK), jnp.float32),
          [pltpu.VMEM((CHUNK, DH), jnp.float32)] * NBUF,
          [pltpu.SemaphoreType.DMA] * NBUF,
          [pltpu.SemaphoreType.DMA] * 2,
          [pltpu.SemaphoreType.DMA] * 2,
          pltpu.VMEM((ZROWS, DH), jnp.float32),
          pltpu.VMEM_SHARED((N_NODES, DH), jnp.float32),
      ],
      compiler_params=pltpu.CompilerParams(use_tc_tiling_on_sc=False,
                                           needs_layout_passes=False),
  )


def _matmul_body(f_ref, w_ref, b_ref, o_ref):
  r = (jnp.dot(f_ref[...], w_ref[...], preferred_element_type=jnp.float32)
       + b_ref[...])
  o_ref[0] = r[:, :DH]
  o_ref[1] = r[:, DH:]


def _matmul(feats, W, b2):
  grid = 10
  rb = N_NODES // grid
  return pl.pallas_call(
      _matmul_body,
      grid=(grid,),
      in_specs=[
          pl.BlockSpec((rb, D), lambda i: (i, 0)),
          pl.BlockSpec((D, D), lambda i: (0, 0)),
          pl.BlockSpec((1, D), lambda i: (0, 0)),
      ],
      out_specs=pl.BlockSpec((NC, rb, DH), lambda i: (0, i, 0)),
      out_shape=jax.ShapeDtypeStruct((NC, N_NODES, DH), jnp.float32),
  )(feats, W, b2)


def _mean_body(e0_ref, es_ref, o_ref):
  v = e0_ref[...] + es_ref[:, 0] + es_ref[:, 1] + es_ref[:, 2]
  v = v * 0.25
  o_ref[...] = jnp.concatenate([v[0], v[1]], axis=-1)


def _mean(e0, es):
  grid = 10
  rb = N_NODES // grid
  return pl.pallas_call(
      _mean_body,
      grid=(grid,),
      in_specs=[
          pl.BlockSpec((NC, rb, DH), lambda i: (0, i, 0)),
          pl.BlockSpec((NC, N_LAYERS, rb, DH), lambda i: (0, 0, i, 0)),
      ],
      out_specs=pl.BlockSpec((rb, D), lambda i: (i, 0)),
      out_shape=jax.ShapeDtypeStruct((N_NODES, D), jnp.float32),
  )(e0, es)


@jax.jit
def kernel(feats, edge_index, edge_weight, W, b):
  pad = E_PER_T_PAD - N_EDGES // NS
  dst = jnp.pad(edge_index[0].reshape(NS, N_EDGES // NS),
                ((0, 0), (0, pad))).reshape(NS, NCHUNK, CHUNK)
  src = jnp.pad(edge_index[1].reshape(NS, N_EDGES // NS),
                ((0, 0), (0, pad))).reshape(NS, NCHUNK, CHUNK)
  idx = jnp.stack([src, dst], axis=2)
  w = jnp.pad(edge_weight.reshape(NS, N_EDGES // NS),
              ((0, 0), (0, pad))).reshape(NS, NCHUNK, CHUNK)

  e0 = _matmul(feats, W, b.reshape(1, D))
  es = _get_sc_layers()(e0, idx, w)
  return _mean(e0, es)


# speedup vs baseline: 1.7793x; 1.7793x over previous
import functools

import jax
import jax.numpy as jnp
from jax import lax
from jax.experimental import pallas as pl
from jax.experimental.pallas import tpu as pltpu
from jax.experimental.pallas import tpu_sc as plsc

N_NODES = 10000
N_EDGES = 320000
D = 128
N_LAYERS = 3

NC = 2
NS = 16
DH = D // NC
CHUNK = 128
E_PER_T_PAD = 20480
NCHUNK = E_PER_T_PAD // CHUNK
NBUF = 4
ROWS_PER_TILE = N_NODES // NS
ZROWS = 125

_BCAST_DNUMS = lax.GatherDimensionNumbers(
    offset_dims=(), collapsed_slice_dims=(0,), start_index_map=(0,))


def _bcast(v16, i, zero):
  idx = jnp.full((16, 1), zero + i, jnp.int32)
  return lax.gather(v16, idx, _BCAST_DNUMS, (1,),
                    mode=lax.GatherScatterMode.PROMISE_IN_BOUNDS)


def _scale_chunk(rows_v, w2_v, p, b):
  @plsc.parallel_loop(0, CHUNK // 16, unroll=2)
  def group(g):
    gbase = pl.multiple_of(g * 16, 16)
    w16 = w2_v[p, b, pl.ds(gbase, 16)]
    zero = g * 0
    for i in range(16):
      wb = _bcast(w16, i, zero)
      e = gbase + i
      for col in range(DH // 16):
        x = rows_v[e, pl.ds(col * 16, 16)]
        rows_v[e, pl.ds(col * 16, 16)] = x * wb


NOUTER = NCHUNK // NBUF


def _sc_layers_body(table_hbm, idx_hbm, w_hbm, out_hbm,
                    sidx_v, w2_v, bufs, gsems, esems, fsems, zbuf_v, acc_sh):
  c = lax.axis_index("c")
  s = lax.axis_index("s")
  zeros16 = jnp.zeros((16,), jnp.float32)

  @plsc.parallel_loop(0, ZROWS, unroll=4)
  def zero_row(r):
    for col in range(DH // 16):
      zbuf_v[r, pl.ds(col * 16, 16)] = zeros16

  def edata_copy(i, p):
    a = pltpu.async_copy(idx_hbm.at[s, pl.ds(i * NBUF, NBUF)],
                         sidx_v.at[p], esems[p])
    bb = pltpu.async_copy(w_hbm.at[s, pl.ds(i * NBUF, NBUF)],
                          w2_v.at[p], fsems[p])
    return a, bb

  def edata_wait(i, p):
    a, bb = pltpu.make_async_copy(idx_hbm.at[s, pl.ds(i * NBUF, NBUF)],
                                  sidx_v.at[p], esems[p]), \
            pltpu.make_async_copy(w_hbm.at[s, pl.ds(i * NBUF, NBUF)],
                                  w2_v.at[p], fsems[p])
    a.wait()
    bb.wait()

  def gather_issue(table, p, b):
    pltpu.async_copy(table.at[c].at[sidx_v.at[p, b, 0]], bufs[b], gsems[b])

  def gather_wait(table, p, b):
    pltpu.make_async_copy(table.at[c].at[sidx_v.at[p, b, 0]],
                          bufs[b], gsems[b]).wait()

  def one_layer(table, lyr):
    for t in range(ROWS_PER_TILE // ZROWS):
      pltpu.sync_copy(zbuf_v, acc_sh.at[pl.ds(s * ROWS_PER_TILE + t * ZROWS,
                                              ZROWS)])
    plsc.subcore_barrier()

    a, bcp = edata_copy(0, 0)
    a.wait()
    bcp.wait()
    for b in range(NBUF):
      gather_issue(table, 0, b)
    edata_copy(1, 1)

    def sub_iter(i, p):
      p1 = 1 - p

      @pl.when(i + 1 < NOUTER)
      def _():
        edata_wait(i + 1, p1)

      for b in range(NBUF):
        gather_wait(table, p, b)
        _scale_chunk(bufs[b], w2_v, p, b)
        pltpu.sync_copy(bufs[b], acc_sh.at[sidx_v.at[p, b, 1]], add=True)

        @pl.when(i + 1 < NOUTER)
        def _():
          gather_issue(table, p1, b)

      @pl.when(i + 2 < NOUTER)
      def _():
        edata_copy(i + 2, p)

    def outer(io, carry):
      sub_iter(io * 2, 0)
      sub_iter(io * 2 + 1, 1)
      return carry

    lax.fori_loop(0, NOUTER // 2, outer, 0)
    plsc.subcore_barrier()

    for t in range(ROWS_PER_TILE // ZROWS):
      r0 = s * ROWS_PER_TILE + t * ZROWS
      pltpu.sync_copy(acc_sh.at[pl.ds(r0, ZROWS)],
                      out_hbm.at[c, lyr, pl.ds(r0, ZROWS)])
    plsc.subcore_barrier()

  one_layer(table_hbm, 0)
  one_layer(out_hbm.at[:, 0], 1)
  one_layer(out_hbm.at[:, 1], 2)


@functools.cache
def _get_sc_layers():
  return pl.kernel(
      _sc_layers_body,
      out_type=jax.ShapeDtypeStruct((NC, N_LAYERS, N_NODES, DH), jnp.float32),
      mesh=plsc.VectorSubcoreMesh(core_axis_name="c", subcore_axis_name="s",
                                  num_cores=NC, num_subcores=NS),
      scratch_types=[
          pltpu.VMEM((2, NBUF, 2, CHUNK), jnp.int32),
          pltpu.VMEM((2, NBUF, CHUNK), jnp.float32),
          [pltpu.VMEM((CHUNK, DH), jnp.float32)] * NBUF,
          [pltpu.SemaphoreType.DMA] * NBUF,
          [pltpu.SemaphoreType.DMA] * 2,
          [pltpu.SemaphoreType.DMA] * 2,
          pltpu.VMEM((ZROWS, DH), jnp.float32),
          pltpu.VMEM_SHARED((N_NODES, DH), jnp.float32),
      ],
      compiler_params=pltpu.CompilerParams(use_tc_tiling_on_sc=False,
                                           needs_layout_passes=False),
  )


def _matmul_body(f_ref, w_ref, b_ref, o_ref):
  r = (jnp.dot(f_ref[...], w_ref[...], preferred_element_type=jnp.float32)
       + b_ref[...])
  o_ref[0] = r[:, :DH]
  o_ref[1] = r[:, DH:]


def _matmul(feats, W, b2):
  grid = 10
  rb = N_NODES // grid
  return pl.pallas_call(
      _matmul_body,
      grid=(grid,),
      in_specs=[
          pl.BlockSpec((rb, D), lambda i: (i, 0)),
          pl.BlockSpec((D, D), lambda i: (0, 0)),
          pl.BlockSpec((1, D), lambda i: (0, 0)),
      ],
      out_specs=pl.BlockSpec((NC, rb, DH), lambda i: (0, i, 0)),
      out_shape=jax.ShapeDtypeStruct((NC, N_NODES, DH), jnp.float32),
  )(feats, W, b2)


def _mean_body(e0_ref, es_ref, o_ref):
  v = e0_ref[...] + es_ref[:, 0] + es_ref[:, 1] + es_ref[:, 2]
  v = v * 0.25
  o_ref[...] = jnp.concatenate([v[0], v[1]], axis=-1)


def _mean(e0, es):
  grid = 10
  rb = N_NODES // grid
  return pl.pallas_call(
      _mean_body,
      grid=(grid,),
      in_specs=[
          pl.BlockSpec((NC, rb, DH), lambda i: (0, i, 0)),
          pl.BlockSpec((NC, N_LAYERS, rb, DH), lambda i: (0, 0, i, 0)),
      ],
      out_specs=pl.BlockSpec((rb, D), lambda i: (i, 0)),
      out_shape=jax.ShapeDtypeStruct((N_NODES, D), jnp.float32),
  )(e0, es)


@jax.jit
def kernel(feats, edge_index, edge_weight, W, b):
  pad = E_PER_T_PAD - N_EDGES // NS
  dst = jnp.pad(edge_index[0].reshape(NS, N_EDGES // NS),
                ((0, 0), (0, pad))).reshape(NS, NCHUNK, CHUNK)
  src = jnp.pad(edge_index[1].reshape(NS, N_EDGES // NS),
                ((0, 0), (0, pad))).reshape(NS, NCHUNK, CHUNK)
  idx = jnp.stack([src, dst], axis=2)
  w = jnp.pad(edge_weight.reshape(NS, N_EDGES // NS),
              ((0, 0), (0, pad))).reshape(NS, NCHUNK, CHUNK)

  e0 = _matmul(feats, W, b.reshape(1, D))
  es = _get_sc_layers()(e0, idx, w)
  return _mean(e0, es)


# speedup vs baseline: 1.8829x; 1.0583x over previous
import functools

import jax
import jax.numpy as jnp
from jax import lax
from jax.experimental import pallas as pl
from jax.experimental.pallas import tpu as pltpu
from jax.experimental.pallas import tpu_sc as plsc

N_NODES = 10000
N_EDGES = 320000
D = 128
N_LAYERS = 3

NC = 2
NS = 16
DH = D // NC
CHUNK = 128
E_PER_T_PAD = 20480
NCHUNK = E_PER_T_PAD // CHUNK
NBUF = 4
ROWS_PER_TILE = N_NODES // NS
ZROWS = 125

_BCAST_DNUMS = lax.GatherDimensionNumbers(
    offset_dims=(), collapsed_slice_dims=(0,), start_index_map=(0,))


def _bcast(v16, i, zero):
  idx = jnp.full((16, 1), zero + i, jnp.int32)
  return lax.gather(v16, idx, _BCAST_DNUMS, (1,),
                    mode=lax.GatherScatterMode.PROMISE_IN_BOUNDS)


RB = 8
LA = 4
RE = 16


def _scale_chunk(rows_v, edata_v, v):
  @plsc.parallel_loop(0, CHUNK // 16, unroll=2)
  def group(g):
    gbase = pl.multiple_of(g * 16, 16)
    w16 = plsc.bitcast(edata_v[v, 2, pl.ds(gbase, 16)], jnp.float32)
    zero = g * 0
    for i in range(16):
      wb = _bcast(w16, i, zero)
      e = gbase + i
      for col in range(DH // 16):
        x = rows_v[e, pl.ds(col * 16, 16)]
        rows_v[e, pl.ds(col * 16, 16)] = x * wb


def _sc_layers_body(table_hbm, idx_hbm, out_hbm,
                    edata_v, bufs, gsems, esems, zbuf_v, acc_sh):
  c = lax.axis_index("c")
  s = lax.axis_index("s")
  zeros16 = jnp.zeros((16,), jnp.float32)

  def edata_issue(j, q):
    pltpu.async_copy(idx_hbm.at[s, j], edata_v.at[q], esems[q])

  def edata_wait(j, q):
    pltpu.make_async_copy(idx_hbm.at[s, j], edata_v.at[q], esems[q]).wait()

  def gather_issue(table, q, b):
    pltpu.async_copy(table.at[c].at[edata_v.at[q, 0]], bufs[b], gsems[b])

  def gather_wait(table, q, b):
    pltpu.make_async_copy(table.at[c].at[edata_v.at[q, 0]],
                          bufs[b], gsems[b]).wait()

  def scatter_sync(q, b):
    pltpu.sync_copy(bufs[b], acc_sh.at[edata_v.at[q, 1]], add=True)

  for t in range(ROWS_PER_TILE // ZROWS):
    r0 = s * ROWS_PER_TILE + t * ZROWS
    pltpu.sync_copy(table_hbm.at[c, pl.ds(r0, ZROWS)], zbuf_v)
    pltpu.sync_copy(zbuf_v, out_hbm.at[c, 0, pl.ds(r0, ZROWS)])
  plsc.subcore_barrier()

  @plsc.parallel_loop(0, ZROWS, unroll=4)
  def zero_zbuf(r):
    for col in range(DH // 16):
      zbuf_v[r, pl.ds(col * 16, 16)] = zeros16

  def one_layer(lyr):
    table = out_hbm.at[:, lyr]
    for t in range(ROWS_PER_TILE // ZROWS):
      pltpu.sync_copy(zbuf_v, acc_sh.at[pl.ds(s * ROWS_PER_TILE + t * ZROWS,
                                              ZROWS)])
    plsc.subcore_barrier()

    for q in range(RE):
      edata_issue(q, q)
    for bq in range(LA):
      edata_wait(bq, bq)
      gather_issue(table, bq, bq)

    def visit(j, v):
      b = v % RB
      b4 = (v + LA) % RB
      v4 = (v + LA) % RE
      v12 = (v + RE - LA) % RE

      gather_wait(table, v, b)
      _scale_chunk(bufs[b], edata_v, v)
      scatter_sync(v, b)

      @pl.when(j + LA < NCHUNK)
      def _():
        edata_wait(j + LA, v4)
        gather_issue(table, v4, b4)

      @pl.when(jnp.logical_and(j >= LA, j + RE - LA < NCHUNK))
      def _():
        edata_issue(j + RE - LA, v12)

    def outer(io, carry):
      for v in range(RE):
        visit(io * RE + v, v)
      return carry

    lax.fori_loop(0, NCHUNK // RE, outer, 0)
    plsc.subcore_barrier()

    for t in range(ROWS_PER_TILE // ZROWS):
      r0 = s * ROWS_PER_TILE + t * ZROWS
      pltpu.sync_copy(acc_sh.at[pl.ds(r0, ZROWS)],
                      out_hbm.at[c, lyr + 1, pl.ds(r0, ZROWS)])
    plsc.subcore_barrier()

  def layer_step(lyr, carry):
    one_layer(lyr)
    return carry

  lax.fori_loop(0, N_LAYERS, layer_step, 0)


@functools.cache
def _get_sc_layers():
  return pl.kernel(
      _sc_layers_body,
      out_type=jax.ShapeDtypeStruct((NC, N_LAYERS + 1, N_NODES, DH),
                                    jnp.float32),
      mesh=plsc.VectorSubcoreMesh(core_axis_name="c", subcore_axis_name="s",
                                  num_cores=NC, num_subcores=NS),
      scratch_types=[
          pltpu.VMEM((RE, 3, CHUNK), jnp.int32),
          [pltpu.VMEM((CHUNK, DH), jnp.float32)] * RB,
          [pltpu.SemaphoreType.DMA] * RB,
          [pltpu.SemaphoreType.DMA] * RE,
          pltpu.VMEM((ZROWS, DH), jnp.float32),
          pltpu.VMEM_SHARED((N_NODES, DH), jnp.float32),
      ],
      compiler_params=pltpu.CompilerParams(use_tc_tiling_on_sc=False,
                                           needs_layout_passes=False),
  )


def _matmul_body(f_ref, w_ref, b_ref, o_ref):
  r = (jnp.dot(f_ref[...], w_ref[...], preferred_element_type=jnp.float32)
       + b_ref[...])
  o_ref[0] = r[:, :DH]
  o_ref[1] = r[:, DH:]


def _matmul(feats, W, b2):
  grid = 10
  rb = N_NODES // grid
  return pl.pallas_call(
      _matmul_body,
      grid=(grid,),
      in_specs=[
          pl.BlockSpec((rb, D), lambda i: (i, 0)),
          pl.BlockSpec((D, D), lambda i: (0, 0)),
          pl.BlockSpec((1, D), lambda i: (0, 0)),
      ],
      out_specs=pl.BlockSpec((NC, rb, DH), lambda i: (0, i, 0)),
      out_shape=jax.ShapeDtypeStruct((NC, N_NODES, DH), jnp.float32),
  )(feats, W, b2)


def _mean_body(es_ref, o_ref):
  v = es_ref[:, 0] + es_ref[:, 1] + es_ref[:, 2] + es_ref[:, 3]
  v = v * 0.25
  o_ref[...] = jnp.concatenate([v[0], v[1]], axis=-1)


def _mean(es):
  grid = 10
  rb = N_NODES // grid
  return pl.pallas_call(
      _mean_body,
      grid=(grid,),
      in_specs=[
          pl.BlockSpec((NC, N_LAYERS + 1, rb, DH), lambda i: (0, 0, i, 0)),
      ],
      out_specs=pl.BlockSpec((rb, D), lambda i: (i, 0)),
      out_shape=jax.ShapeDtypeStruct((N_NODES, D), jnp.float32),
  )(es)


@jax.jit
def kernel(feats, edge_index, edge_weight, W, b):
  pad = E_PER_T_PAD - N_EDGES // NS
  dst = jnp.pad(edge_index[0].reshape(NS, N_EDGES // NS),
                ((0, 0), (0, pad))).reshape(NS, NCHUNK, CHUNK)
  src = jnp.pad(edge_index[1].reshape(NS, N_EDGES // NS),
                ((0, 0), (0, pad))).reshape(NS, NCHUNK, CHUNK)
  wbits = lax.bitcast_convert_type(
      jnp.pad(edge_weight.reshape(NS, N_EDGES // NS),
              ((0, 0), (0, pad))).reshape(NS, NCHUNK, CHUNK), jnp.int32)
  idx = jnp.stack([src, dst, wbits], axis=2)

  e0 = _matmul(feats, W, b.reshape(1, D))
  es = _get_sc_layers()(e0, idx)
  return _mean(es)
